# baseline (speedup 1.0000x reference)
import jax
import jax.numpy as jnp
from jax.experimental import pallas as pl


def _add_body(x_ref, e_ref, o_ref):
    o_ref[...] = x_ref[...] + e_ref[...][None]


def kernel(x, emb_table):
    B, T, D = x.shape
    BT = 1024
    BB = 2
    T0 = 6144
    def part(t_lo, t_hi):
        nt = (t_hi - t_lo) // BT
        off = t_lo // BT
        return pl.pallas_call(
            _add_body,
            grid=(nt, B // BB),
            in_specs=[
                pl.BlockSpec((BB, BT, D), lambda i, j: (j, i + off, 0)),
                pl.BlockSpec((BT, D), lambda i, j: (i + off, 0)),
            ],
            out_specs=pl.BlockSpec((BB, BT, D), lambda i, j: (j, i, 0)),
            out_shape=jax.ShapeDtypeStruct((B, t_hi - t_lo, D), x.dtype),
        )(x, emb_table)
    return jnp.concatenate([part(0, T0), part(T0, T)], axis=1)
